# baseline (speedup 1.0000x reference)
import functools

import jax
import jax.numpy as jnp
from jax.experimental import pallas as pl
from jax.experimental.pallas import tpu as pltpu


def _halves_kernel(emb_hbm, out_hbm, buf, in_sems, out_sems, *,
                   scale, chunks):
    def in_copy(i):
        base, rows = chunks[i]
        return pltpu.make_async_copy(
            emb_hbm.at[pl.ds(base, rows)],
            buf.at[i, pl.ds(0, rows)],
            in_sems.at[i],
        )

    def out_copy(i):
        base, rows = chunks[i]
        return pltpu.make_async_copy(
            buf.at[i, pl.ds(0, rows)],
            out_hbm.at[pl.ds(base, rows)],
            out_sems.at[i],
        )

    n = len(chunks)
    for i in range(n):
        in_copy(i).start()
    for i in range(n):
        in_copy(i).wait()
        rows = chunks[i][1]
        buf[i, pl.ds(0, rows)] = (buf[i, pl.ds(0, rows)] * scale
                                  ).astype(buf.dtype)
        out_copy(i).start()
    for i in range(n):
        out_copy(i).wait()


def kernel(x, embedding):
    max_seq_len, dim = embedding.shape
    seq_len = x.shape[1]
    if seq_len > max_seq_len:
        raise ValueError(f"seq_len={seq_len} exceeds max_seq_len={max_seq_len}")
    dtype = embedding.dtype
    itemsize = jnp.dtype(dtype).itemsize
    row_bytes = dim * itemsize

    n_chunks = 2
    chunk_rows = -(-seq_len // n_chunks)
    chunks = []
    base = 0
    while base < seq_len:
        rows = min(chunk_rows, seq_len - base)
        chunks.append((base, rows))
        base += rows

    vmem_bytes = len(chunks) * chunk_rows * row_bytes
    vmem_limit = int(min(110 * 1024 * 1024,
                         max(16 * 1024 * 1024, vmem_bytes + 4 * 1024 * 1024)))

    return pl.pallas_call(
        functools.partial(_halves_kernel, scale=float(dim) ** -0.5,
                          chunks=chunks),
        out_shape=jax.ShapeDtypeStruct((seq_len, dim), dtype),
        in_specs=[pl.BlockSpec(memory_space=pl.ANY)],
        out_specs=pl.BlockSpec(memory_space=pl.ANY),
        scratch_shapes=[
            pltpu.VMEM((len(chunks), chunk_rows, dim), dtype),
            pltpu.SemaphoreType.DMA((len(chunks),)),
            pltpu.SemaphoreType.DMA((len(chunks),)),
        ],
        compiler_params=pltpu.CompilerParams(
            vmem_limit_bytes=vmem_limit,
        ),
    )(embedding)
